# baseline (speedup 1.0000x reference)
import functools

import jax
import jax.numpy as jnp
from jax import lax
from jax.experimental import pallas as pl
from jax.experimental.pallas import tpu as pltpu
from jax.experimental.pallas import tpu_sc as plsc

D_MODEL = 2048
BATCH = 4
SEQ = 2048
B_TOTAL = BATCH * SEQ
NUM_CORES = 2
NUM_SUBCORES = 16
NW = NUM_CORES * NUM_SUBCORES
B_PER_W = B_TOTAL // NW
CHUNK = 16
N_CHUNKS = B_PER_W // CHUNK
NBUF = 3


def _make_gather_kernel():
    mesh = plsc.VectorSubcoreMesh(core_axis_name="c", subcore_axis_name="s")

    @functools.partial(
        pl.kernel,
        mesh=mesh,
        out_type=jax.ShapeDtypeStruct((B_TOTAL, D_MODEL), jnp.float32),
        scratch_types=[pltpu.VMEM((N_CHUNKS, CHUNK), jnp.int32)]
        + [pltpu.VMEM((CHUNK, D_MODEL), jnp.float32) for _ in range(NBUF)]
        + [pltpu.SemaphoreType.DMA for _ in range(2 * NBUF)],
    )
    def gather_kernel(x_hbm, table_hbm, out_hbm, idx_v, *bufs_and_sems):
        bufs = bufs_and_sems[:NBUF]
        gsems = bufs_and_sems[NBUF:2 * NBUF]
        wsems = bufs_and_sems[2 * NBUF:]
        wid = lax.axis_index("s") * NUM_CORES + lax.axis_index("c")
        base = wid * B_PER_W
        pltpu.sync_copy(x_hbm.at[wid], idx_v)

        def start_gather(j, b):
            return pltpu.async_copy(table_hbm.at[idx_v.at[j]], bufs[b], gsems[b])

        def start_write(j, b):
            return pltpu.async_copy(
                bufs[b], out_hbm.at[pl.ds(base + j * CHUNK, CHUNK)], wsems[b]
            )

        gh = {}
        wh = {}
        waited = set()
        for j in range(NBUF):
            gh[j] = start_gather(j, j)
        for j in range(N_CHUNKS):
            gh[j].wait()
            wh[j] = start_write(j, j % NBUF)
            m = j - 1
            if m >= 0 and m + NBUF < N_CHUNKS:
                wh[m].wait()
                waited.add(m)
                gh[m + NBUF] = start_gather(m + NBUF, m % NBUF)
        for j in range(N_CHUNKS):
            if j not in waited:
                wh[j].wait()

    return gather_kernel


_gather = _make_gather_kernel()


def kernel(x, embedding):
    xw = x.reshape(-1).astype(jnp.int32).reshape(NW, N_CHUNKS, CHUNK)
    out = _gather(xw, embedding)
    return out.reshape(x.shape[0], x.shape[1], D_MODEL)


# speedup vs baseline: 1.0245x; 1.0245x over previous
import functools

import jax
import jax.numpy as jnp
from jax import lax
from jax.experimental import pallas as pl
from jax.experimental.pallas import tpu as pltpu
from jax.experimental.pallas import tpu_sc as plsc

D_MODEL = 2048
BATCH = 4
SEQ = 2048
B_TOTAL = BATCH * SEQ
NUM_CORES = 2
NUM_SUBCORES = 16
NW = NUM_CORES * NUM_SUBCORES
B_PER_W = B_TOTAL // NW
W_PER_ROW = SEQ // B_PER_W
CHUNK = 16
N_CHUNKS = B_PER_W // CHUNK
NBUF = 3


def _make_gather_kernel():
    mesh = plsc.VectorSubcoreMesh(core_axis_name="c", subcore_axis_name="s")

    @functools.partial(
        pl.kernel,
        mesh=mesh,
        out_type=jax.ShapeDtypeStruct((B_TOTAL, D_MODEL), jnp.float32),
        scratch_types=[pltpu.VMEM((B_PER_W,), jnp.int32)]
        + [pltpu.VMEM((CHUNK, D_MODEL), jnp.float32) for _ in range(NBUF)]
        + [pltpu.SemaphoreType.DMA for _ in range(2 * NBUF)],
    )
    def gather_kernel(x_hbm, table_hbm, out_hbm, idx_v, *bufs_and_sems):
        bufs = bufs_and_sems[:NBUF]
        gsems = bufs_and_sems[NBUF:2 * NBUF]
        wsems = bufs_and_sems[2 * NBUF:]
        wid = lax.axis_index("s") * NUM_CORES + lax.axis_index("c")
        base = wid * B_PER_W
        pltpu.sync_copy(
            x_hbm.at[wid // W_PER_ROW, pl.ds((wid % W_PER_ROW) * B_PER_W, B_PER_W)],
            idx_v,
        )

        def start_gather(j, b):
            return pltpu.async_copy(
                table_hbm.at[idx_v.at[pl.ds(j * CHUNK, CHUNK)]], bufs[b], gsems[b]
            )

        def start_write(j, b):
            return pltpu.async_copy(
                bufs[b], out_hbm.at[pl.ds(base + j * CHUNK, CHUNK)], wsems[b]
            )

        gh = {}
        wh = {}
        for j in range(NBUF):
            gh[j] = start_gather(j, j)
        for j in range(N_CHUNKS):
            b = j % NBUF
            gh[j].wait()
            wh[j] = start_write(j, b)
            jn = j + NBUF
            if jn < N_CHUNKS:
                wh[j].wait()
                gh[jn] = start_gather(jn, b)
        for j in range(N_CHUNKS - NBUF, N_CHUNKS):
            wh[j].wait()

    return gather_kernel


_gather = _make_gather_kernel()


def kernel(x, embedding):
    out = _gather(x.astype(jnp.int32), embedding)
    return out.reshape(x.shape[0], x.shape[1], D_MODEL)


# speedup vs baseline: 1.0351x; 1.0104x over previous
import functools

import jax
import jax.numpy as jnp
from jax import lax
from jax.experimental import pallas as pl
from jax.experimental.pallas import tpu as pltpu
from jax.experimental.pallas import tpu_sc as plsc

D_MODEL = 2048
BATCH = 4
SEQ = 2048
B_TOTAL = BATCH * SEQ
NUM_CORES = 2
NUM_SUBCORES = 16
NW = NUM_CORES * NUM_SUBCORES
B_PER_W = B_TOTAL // NW
W_PER_ROW = SEQ // B_PER_W
CHUNK = 8
N_CHUNKS = B_PER_W // CHUNK
NBUF = 7


def _make_gather_kernel():
    mesh = plsc.VectorSubcoreMesh(core_axis_name="c", subcore_axis_name="s")

    @functools.partial(
        pl.kernel,
        mesh=mesh,
        out_type=jax.ShapeDtypeStruct((B_TOTAL, D_MODEL), jnp.float32),
        scratch_types=[pltpu.VMEM((B_PER_W,), jnp.int32)]
        + [pltpu.VMEM((CHUNK, D_MODEL), jnp.float32) for _ in range(NBUF)]
        + [pltpu.SemaphoreType.DMA for _ in range(2 * NBUF)],
    )
    def gather_kernel(x_hbm, table_hbm, out_hbm, idx_v, *bufs_and_sems):
        bufs = bufs_and_sems[:NBUF]
        gsems = bufs_and_sems[NBUF:2 * NBUF]
        wsems = bufs_and_sems[2 * NBUF:]
        wid = lax.axis_index("s") * NUM_CORES + lax.axis_index("c")
        base = wid * B_PER_W
        pltpu.sync_copy(
            x_hbm.at[wid // W_PER_ROW, pl.ds((wid % W_PER_ROW) * B_PER_W, B_PER_W)],
            idx_v,
        )

        def start_gather(j, b):
            return pltpu.async_copy(
                table_hbm.at[idx_v.at[pl.ds(j * CHUNK, CHUNK)]], bufs[b], gsems[b]
            )

        def start_write(j, b):
            return pltpu.async_copy(
                bufs[b], out_hbm.at[pl.ds(base + j * CHUNK, CHUNK)], wsems[b]
            )

        WLAG = 2
        gh = {}
        wh = {}
        waited = set()
        for j in range(NBUF):
            gh[j] = start_gather(j, j)
        for j in range(N_CHUNKS):
            gh[j].wait()
            wh[j] = start_write(j, j % NBUF)
            m = j - WLAG
            if m >= 0 and m + NBUF < N_CHUNKS:
                wh[m].wait()
                waited.add(m)
                gh[m + NBUF] = start_gather(m + NBUF, m % NBUF)
        for j in range(N_CHUNKS):
            if j not in waited:
                wh[j].wait()

    return gather_kernel


_gather = _make_gather_kernel()


def kernel(x, embedding):
    out = _gather(x.astype(jnp.int32), embedding)
    return out.reshape(x.shape[0], x.shape[1], D_MODEL)
